# baseline (speedup 1.0000x reference)
import functools

import jax
import jax.numpy as jnp
from jax import lax
from jax.experimental import pallas as pl
from jax.experimental.pallas import tpu as pltpu
from jax.experimental.pallas import tpu_sc as plsc

_NC = 2
_NS = 16
_NW = _NC * _NS
_L = 16

_B = 4096
_T = 200
_N = 64
_R = _B * _T
_RW = _R // _NW
_IDXW = 100
_CB = 2
_CR = _CB * _T
_IDX_ROWS = _CR // _IDXW
_CHUNKS = _RW // _CR
_GRP = _N // _L
_UNROLL = 4


def _emb_body(tok_hbm, idx_hbm, pos_hbm, out_hbm,
              idx0, idx1, rows0, rows1, pos_v,
              sidx0, sidx1, sg0, sg1, so0, so1):
    wid = lax.axis_index("s") * _NC + lax.axis_index("c")
    idx_v = (idx0, idx1)
    rows_v = (rows0, rows1)
    sidx = (sidx0, sidx1)
    sg = (sg0, sg1)
    so = (so0, so1)

    pltpu.sync_copy(pos_hbm.at[pl.ds(0, _T)], pos_v)

    def idx_slice(c):
        irow = wid * (_RW // _IDXW) + c * _IDX_ROWS
        return idx_hbm.at[pl.ds(irow, _IDX_ROWS)]

    def out_slice(c):
        rowbase = wid * _RW + c * _CR
        return out_hbm.at[pl.ds(rowbase, _CR)]

    def start_idx(c, b):
        pltpu.make_async_copy(idx_slice(c), idx_v[b], sidx[b]).start()

    def wait_idx(c, b):
        pltpu.make_async_copy(idx_slice(c), idx_v[b], sidx[b]).wait()

    def start_gathers(b):
        for g in range(_IDX_ROWS):
            pltpu.make_async_copy(
                tok_hbm.at[idx_v[b].at[g]],
                rows_v[b].at[pl.ds(g * _IDXW, _IDXW)],
                sg[b],
            ).start()

    def wait_gathers(b):
        pltpu.make_async_copy(tok_hbm.at[pl.ds(0, _CR)], rows_v[b], sg[b]).wait()

    def start_out(c, b):
        pltpu.make_async_copy(rows_v[b], out_slice(c), so[b]).start()

    def wait_out(c, b):
        pltpu.make_async_copy(rows_v[b], out_slice(c), so[b]).wait()

    def add_pos(b):
        rv = rows_v[b]

        def add_body(r0, acc):
            for dr in range(_UNROLL):
                r = r0 * _UNROLL + dr
                for g in range(_GRP):
                    sl = pl.ds(g * _L, _L)
                    p = pos_v[r, sl]
                    for rep in range(_CB):
                        rr = r + rep * _T
                        rv[rr, sl] = rv[rr, sl] + p
            return acc

        lax.fori_loop(0, _T // _UNROLL, add_body, 0, unroll=2)

    start_idx(0, 0)

    def pair_body(c2, acc):
        for par in range(2):
            c = c2 * 2 + par
            b = par
            o = 1 - par
            wait_idx(c, b)

            @pl.when(c >= 2)
            def _():
                wait_out(c - 2, b)

            start_gathers(b)

            @pl.when(c >= 1)
            def _():
                wait_gathers(o)

            @pl.when(c + 1 < _CHUNKS)
            def _():
                start_idx(c + 1, o)

            @pl.when(c >= 1)
            def _():
                add_pos(o)
                start_out(c - 1, o)
        return acc

    lax.fori_loop(0, _CHUNKS // 2, pair_body, 0)

    bl = (_CHUNKS - 1) % 2
    wait_gathers(bl)
    add_pos(bl)
    start_out(_CHUNKS - 1, bl)
    wait_out(_CHUNKS - 2, 1 - bl)
    wait_out(_CHUNKS - 1, bl)


@functools.partial(jax.jit, static_argnums=())
def kernel(idx, tok_emb, pos_emb):
    b, t = idx.shape
    n = tok_emb.shape[1]
    idx2d = idx.astype(jnp.int32).reshape(-1, _IDXW)
    mesh = plsc.VectorSubcoreMesh(core_axis_name="c", subcore_axis_name="s")
    emb = pl.kernel(
        _emb_body,
        out_type=jax.ShapeDtypeStruct((_R, _N), jnp.float32),
        mesh=mesh,
        scratch_types=[
            pltpu.VMEM((_IDX_ROWS, _IDXW), jnp.int32),
            pltpu.VMEM((_IDX_ROWS, _IDXW), jnp.int32),
            pltpu.VMEM((_CR, _N), jnp.float32),
            pltpu.VMEM((_CR, _N), jnp.float32),
            pltpu.VMEM((_T, _N), jnp.float32),
            pltpu.SemaphoreType.DMA,
            pltpu.SemaphoreType.DMA,
            pltpu.SemaphoreType.DMA,
            pltpu.SemaphoreType.DMA,
            pltpu.SemaphoreType.DMA,
            pltpu.SemaphoreType.DMA,
        ],
        compiler_params=pltpu.CompilerParams(use_tc_tiling_on_sc=False),
    )
    out = emb(tok_emb, idx2d, pos_emb)
    return out.reshape(b, t, n)
